# baseline (speedup 1.0000x reference)
import functools

import jax
import jax.numpy as jnp
from jax import lax
from jax.experimental import pallas as pl
from jax.experimental.pallas import tpu as pltpu
from jax.experimental.pallas import tpu_sc as plsc

_NUM_TYPES = 87
_TPAD = 128
_F = 128
_N = 100000
_CH = 128
_TCB = 800


def _fuse_body(extra_ref, onehot_ref, econf_ref, wconf_ref, w1_ref, out_ref):
    cfg = lax.dot_general(
        econf_ref[...], wconf_ref[...], (((1,), (1,)), ((), ())),
        preferred_element_type=jnp.float32)
    cat = jnp.concatenate([extra_ref[...], onehot_ref[...], cfg], axis=1)
    out_ref[...] = lax.dot_general(
        cat, w1_ref[...], (((1,), (1,)), ((), ())),
        preferred_element_type=jnp.float32)


def _fused_table(extra, onehot, econf, wconf, w1):
    return pl.pallas_call(
        _fuse_body,
        out_shape=jax.ShapeDtypeStruct((_TPAD, _F), jnp.float32),
    )(extra, onehot, econf, wconf, w1)


def _tc_gather_body(ids_ref, tab_ref, out_ref):
    ids = ids_ref[0, 0, :]
    cols = lax.broadcasted_iota(jnp.int32, (_TCB, _TPAD), 1)
    onehot = (ids[:, None] == cols).astype(jnp.float32)
    out_ref[...] = lax.dot_general(
        onehot, tab_ref[...], (((1,), (0,)), ((), ())),
        preferred_element_type=jnp.float32)


def _tc_gather(ids3d, table):
    nb = _N // _TCB
    return pl.pallas_call(
        _tc_gather_body,
        grid=(nb,),
        in_specs=[
            pl.BlockSpec((1, 1, _TCB), lambda i: (i, 0, 0)),
            pl.BlockSpec((_TPAD, _F), lambda i: (0, 0)),
        ],
        out_specs=pl.BlockSpec((_TCB, _F), lambda i: (i, 0)),
        out_shape=jax.ShapeDtypeStruct((_N, _F), jnp.float32),
    )(ids3d, table)


@functools.cache
def _make_sc_gather():
    info = plsc.get_sparse_core_info()
    nc, ns = info.num_cores, info.num_subcores
    nw = nc * ns
    b_per_w = ((_N + nw - 1) // nw + _CH - 1) // _CH * _CH
    n_chunks = b_per_w // _CH

    mesh = plsc.VectorSubcoreMesh(core_axis_name="c", subcore_axis_name="s")

    @functools.partial(
        pl.kernel,
        out_type=jax.ShapeDtypeStruct((_N, _F), jnp.float32),
        mesh=mesh,
        scratch_types=[
            pltpu.VMEM((b_per_w,), jnp.int32),
            pltpu.VMEM((_CH, _F), jnp.float32),
            pltpu.VMEM((_CH, _F), jnp.float32),
            pltpu.VMEM_SHARED((_TPAD, _F), jnp.float32),
            pltpu.SemaphoreType.DMA,
            pltpu.SemaphoreType.DMA,
            pltpu.SemaphoreType.DMA,
            pltpu.SemaphoreType.DMA,
            pltpu.SemaphoreType.DMA,
        ],
    )
    def gather_k(idx_hbm, table_hbm, out_hbm, idx_v, buf_a, buf_b, tab_s,
                 isem, gsem_a, gsem_b, ssem_a, ssem_b):
        sid = lax.axis_index("s")
        wid = sid * nc + lax.axis_index("c")
        start = wid * b_per_w

        @pl.when(sid == 0)
        def _():
            pltpu.sync_copy(table_hbm, tab_s)
        plsc.subcore_barrier()

        s = [pl.multiple_of(jnp.minimum(start + i * _CH, _N - _CH), 32)
             for i in range(n_chunks)]

        ih = [pltpu.async_copy(idx_hbm.at[pl.ds(s[i], _CH)],
                               idx_v.at[pl.ds(i * _CH, _CH)], isem)
              for i in range(n_chunks)]
        for h in ih:
            h.wait()

        bufs = (buf_a, buf_b)
        gsems = (gsem_a, gsem_b)
        ssems = (ssem_a, ssem_b)
        gh = [None] * n_chunks
        sh = [None] * n_chunks
        for i in range(n_chunks):
            b = i % 2
            if i >= 2:
                sh[i - 2].wait()
            gh[i] = pltpu.async_copy(
                tab_s.at[idx_v.at[pl.ds(i * _CH, _CH)]], bufs[b], gsems[b])
            if i >= 1:
                pb = (i - 1) % 2
                gh[i - 1].wait()
                sh[i - 1] = pltpu.async_copy(
                    bufs[pb], out_hbm.at[pl.ds(s[i - 1], _CH)], ssems[pb])
        last = n_chunks - 1
        gh[last].wait()
        sh[last] = pltpu.async_copy(
            bufs[last % 2], out_hbm.at[pl.ds(s[last], _CH)], ssems[last % 2])
        sh[last - 1].wait()
        sh[last].wait()

    return gather_k


def kernel(atom_types, extra_table, W_onehot, electron_config, W_config, W1):
    pad = ((0, _TPAD - _NUM_TYPES), (0, 0))
    table = _fused_table(
        jnp.pad(extra_table, pad), jnp.pad(W_onehot, pad),
        jnp.pad(electron_config, pad), W_config, W1)
    ids = atom_types.astype(jnp.int32)
    out1 = _make_sc_gather()(ids, table)
    out2 = _tc_gather(ids.reshape(_N // _TCB, 1, _TCB), table)
    return out1, out2


# speedup vs baseline: 1.8488x; 1.8488x over previous
import functools

import jax
import jax.numpy as jnp
from jax import lax
from jax.experimental import pallas as pl
from jax.experimental.pallas import tpu as pltpu
from jax.experimental.pallas import tpu_sc as plsc

_NUM_TYPES = 87
_F = 128
_N = 100000
_CH = 128


def _fuse_body(extra_ref, onehot_ref, econf_ref, wconf_ref, w1_ref, out_ref):
    cfg = lax.dot_general(
        econf_ref[...], wconf_ref[...], (((1,), (1,)), ((), ())),
        preferred_element_type=jnp.float32)
    cat = jnp.concatenate([extra_ref[...], onehot_ref[...], cfg], axis=1)
    out_ref[...] = lax.dot_general(
        cat, w1_ref[...], (((1,), (1,)), ((), ())),
        preferred_element_type=jnp.float32)


def _fused_table(extra, onehot, econf, wconf, w1):
    return pl.pallas_call(
        _fuse_body,
        out_shape=jax.ShapeDtypeStruct((_NUM_TYPES, _F), jnp.float32),
    )(extra, onehot, econf, wconf, w1)


@functools.cache
def _make_gather():
    info = plsc.get_sparse_core_info()
    nc, ns = info.num_cores, info.num_subcores
    nw = nc * ns
    b_per_w = ((_N + nw - 1) // nw + _CH - 1) // _CH * _CH
    n_chunks = b_per_w // _CH

    mesh = plsc.VectorSubcoreMesh(core_axis_name="c", subcore_axis_name="s")

    @functools.partial(
        pl.kernel,
        out_type=(jax.ShapeDtypeStruct((_N, _F), jnp.float32),
                  jax.ShapeDtypeStruct((_N, _F), jnp.float32)),
        mesh=mesh,
        scratch_types=[
            pltpu.VMEM((b_per_w,), jnp.int32),
            pltpu.VMEM((_CH, _F), jnp.float32),
            pltpu.VMEM((_CH, _F), jnp.float32),
            pltpu.VMEM((_CH, _F), jnp.float32),
            pltpu.VMEM((_CH, _F), jnp.float32),
            pltpu.VMEM_SHARED((_NUM_TYPES, _F), jnp.float32),
            pltpu.SemaphoreType.DMA,
            pltpu.SemaphoreType.DMA,
            pltpu.SemaphoreType.DMA,
            pltpu.SemaphoreType.DMA,
            pltpu.SemaphoreType.DMA,
            pltpu.SemaphoreType.DMA,
            pltpu.SemaphoreType.DMA,
            pltpu.SemaphoreType.DMA,
            pltpu.SemaphoreType.DMA,
        ],
    )
    def gather_k(idx_hbm, table_hbm, out_hbm, out2_hbm, idx_v,
                 buf_a, buf_b, buf_c, buf_d, tab_s,
                 isem, gsem_a, gsem_b, gsem_c, gsem_d,
                 ssem_a, ssem_b, ssem_c, ssem_d):
        sid = lax.axis_index("s")
        wid = sid * nc + lax.axis_index("c")
        start = wid * b_per_w

        @pl.when(sid == 0)
        def _():
            pltpu.sync_copy(table_hbm, tab_s)
        plsc.subcore_barrier()

        s = [pl.multiple_of(jnp.minimum(start + i * _CH, _N - _CH), 32)
             for i in range(n_chunks)]

        ih = [pltpu.async_copy(idx_hbm.at[pl.ds(s[i], _CH)],
                               idx_v.at[pl.ds(i * _CH, _CH)], isem)
              for i in range(n_chunks)]
        for h in ih:
            h.wait()

        nbuf = 4
        bufs = (buf_a, buf_b, buf_c, buf_d)
        gsems = (gsem_a, gsem_b, gsem_c, gsem_d)
        ssems = (ssem_a, ssem_b, ssem_c, ssem_d)
        gh = [None] * n_chunks
        sh = [None] * n_chunks
        sh2 = [None] * n_chunks

        def issue_stores(i):
            b = i % nbuf
            sh[i] = pltpu.async_copy(
                bufs[b], out_hbm.at[pl.ds(s[i], _CH)], ssems[b])
            sh2[i] = pltpu.async_copy(
                bufs[b], out2_hbm.at[pl.ds(s[i], _CH)], ssems[b])

        for i in range(n_chunks):
            b = i % nbuf
            if i >= nbuf:
                sh[i - nbuf].wait()
                sh2[i - nbuf].wait()
            gh[i] = pltpu.async_copy(
                tab_s.at[idx_v.at[pl.ds(i * _CH, _CH)]], bufs[b], gsems[b])
            if i >= 1:
                gh[i - 1].wait()
                issue_stores(i - 1)
        last = n_chunks - 1
        gh[last].wait()
        issue_stores(last)
        for i in range(max(0, n_chunks - nbuf), n_chunks):
            sh[i].wait()
            sh2[i].wait()

    return gather_k


def kernel(atom_types, extra_table, W_onehot, electron_config, W_config, W1):
    table = _fused_table(extra_table, W_onehot, electron_config, W_config, W1)
    out, out2 = _make_gather()(atom_types.astype(jnp.int32), table)
    return out, out2
